# initial kernel scaffold (unmeasured)
import jax
import jax.numpy as jnp
from jax import lax
from jax.experimental import pallas as pl
from jax.experimental.pallas import tpu as pltpu


def kernel(
    x,
):
    def body(*refs):
        pass

    out_shape = jax.ShapeDtypeStruct(..., jnp.float32)
    return pl.pallas_call(body, out_shape=out_shape)(...)


# baseline (device time: 113248 ns/iter reference)
import jax
import jax.numpy as jnp
from jax import lax
from jax.experimental import pallas as pl
from jax.experimental.pallas import tpu as pltpu

N_DEV = 4


def kernel(x):
    m_per, n = x.shape
    m_total = N_DEV * m_per

    def body(x_ref, out_ref, gath_ref, send_sems, recv_sems):
        my = lax.axis_index("i")
        left = (my - 1) % N_DEV
        right = (my + 1) % N_DEV

        barrier_sem = pltpu.get_barrier_semaphore()
        for nbr in [left, right]:
            pl.semaphore_signal(
                barrier_sem, inc=1,
                device_id=(nbr,), device_id_type=pl.DeviceIdType.MESH,
            )
        pl.semaphore_wait(barrier_sem, 2)

        gath_ref[pl.ds(my * m_per, m_per), :] = x_ref[...].astype(jnp.bfloat16)

        for h in range(N_DEV - 1):
            origin_send = (my - h) % N_DEV
            rdma = pltpu.make_async_remote_copy(
                src_ref=gath_ref.at[pl.ds(origin_send * m_per, m_per)],
                dst_ref=gath_ref.at[pl.ds(origin_send * m_per, m_per)],
                send_sem=send_sems.at[h],
                recv_sem=recv_sems.at[h],
                device_id=(right,),
                device_id_type=pl.DeviceIdType.MESH,
            )
            rdma.start()
            rdma.wait()

        val = gath_ref[...]
        row = lax.broadcasted_iota(jnp.int32, (m_total, n), 0)
        k = 2
        while k <= m_total:
            j = k // 2
            while j >= 1:
                down = jnp.concatenate([val[j:], val[:j]], axis=0)
                up = jnp.concatenate([val[m_total - j:], val[:m_total - j]], axis=0)
                upper = (row & j) != 0
                pval = jnp.where(upper, up, down)
                asc = (row & k) == 0
                take_min = upper != asc
                val = jnp.where(
                    take_min, jnp.minimum(val, pval), jnp.maximum(val, pval)
                )
                j //= 2
            k *= 2
        gath_ref[...] = val

        out_ref[...] = gath_ref[pl.ds(my * m_per, m_per), :].astype(jnp.float32)

    return pl.pallas_call(
        body,
        out_shape=jax.ShapeDtypeStruct((m_per, n), jnp.float32),
        in_specs=[pl.BlockSpec(memory_space=pltpu.VMEM)],
        out_specs=pl.BlockSpec(memory_space=pltpu.VMEM),
        scratch_shapes=[
            pltpu.VMEM((m_total, n), jnp.bfloat16),
            pltpu.SemaphoreType.DMA((N_DEV - 1,)),
            pltpu.SemaphoreType.DMA((N_DEV - 1,)),
        ],
        compiler_params=pltpu.CompilerParams(collective_id=0),
    )(x)


# device time: 51602 ns/iter; 2.1946x vs baseline; 2.1946x over previous
import jax
import jax.numpy as jnp
from jax import lax
from jax.experimental import pallas as pl
from jax.experimental.pallas import tpu as pltpu

N_DEV = 4


def _ce_stage(val, j, row, asc):
    length = val.shape[0]
    down = jnp.concatenate([val[j:], val[:j]], axis=0)
    up = jnp.concatenate([val[length - j:], val[:length - j]], axis=0)
    upper = (row & j) != 0
    pval = jnp.where(upper, up, down)
    take_min = upper != asc
    return jnp.where(take_min, jnp.minimum(val, pval), jnp.maximum(val, pval))


def _bitonic_merge(val, k, row, asc):
    j = k // 2
    while j >= 1:
        val = _ce_stage(val, j, row, asc)
        j //= 2
    return val


def kernel(x):
    m_per, n = x.shape
    m_total = N_DEV * m_per

    def body(x_ref, out_ref, gath_ref, send_sems, recv_sems):
        my = lax.axis_index("i")
        left = (my - 1) % N_DEV
        right = (my + 1) % N_DEV

        val = x_ref[...].astype(jnp.bfloat16)
        lrow = lax.broadcasted_iota(jnp.int32, (m_per, n), 0) + my * m_per
        k = 2
        while k <= m_per:
            val = _bitonic_merge(val, k, lrow, (lrow & k) == 0)
            k *= 2
        gath_ref[pl.ds(my * m_per, m_per), :] = val

        barrier_sem = pltpu.get_barrier_semaphore()
        for nbr in [left, right]:
            pl.semaphore_signal(
                barrier_sem, inc=1,
                device_id=(nbr,), device_id_type=pl.DeviceIdType.MESH,
            )
        pl.semaphore_wait(barrier_sem, 2)

        def copy(src_slot, dst_slot, sem_idx, target):
            return pltpu.make_async_remote_copy(
                src_ref=gath_ref.at[pl.ds(src_slot * m_per, m_per)],
                dst_ref=gath_ref.at[pl.ds(dst_slot * m_per, m_per)],
                send_sem=send_sems.at[sem_idx],
                recv_sem=recv_sems.at[sem_idx],
                device_id=(target,),
                device_id_type=pl.DeviceIdType.MESH,
            )

        send_r = copy(my, my, 0, right)
        send_r.start()
        send_l = copy(my, my, 1, left)
        send_l.start()

        recv_l = copy(left, left, 0, left)
        recv_l.wait_recv()
        fwd = copy(left, left, 2, right)
        fwd.start()
        recv_r = copy(right, right, 1, right)
        recv_r.wait_recv()

        p0 = my - (my % 2)
        q0 = (p0 + 2) % N_DEV
        half = 2 * m_per
        hrow = lax.broadcasted_iota(jnp.int32, (half, n), 0)
        own_asc = p0 < 2
        own = gath_ref[pl.ds(p0 * m_per, half), :]
        own = _bitonic_merge(own, half, hrow, own_asc)

        recv_d = copy((my - 2) % N_DEV, (my - 2) % N_DEV, 2, left)
        recv_d.wait_recv()
        other = gath_ref[pl.ds(q0 * m_per, half), :]
        other = _bitonic_merge(other, half, hrow, jnp.logical_not(own_asc))

        block_a = jnp.where(own_asc, own, other)
        block_b = jnp.where(own_asc, other, own)
        val = jnp.concatenate([block_a, block_b], axis=0)
        frow = lax.broadcasted_iota(jnp.int32, (m_total, n), 0)
        val = _bitonic_merge(val, m_total, frow, True)

        gath_ref[...] = val
        out_ref[...] = gath_ref[pl.ds(my * m_per, m_per), :].astype(jnp.float32)

        send_r.wait_send()
        send_l.wait_send()
        fwd.wait_send()

    return pl.pallas_call(
        body,
        out_shape=jax.ShapeDtypeStruct((m_per, n), jnp.float32),
        in_specs=[pl.BlockSpec(memory_space=pltpu.VMEM)],
        out_specs=pl.BlockSpec(memory_space=pltpu.VMEM),
        scratch_shapes=[
            pltpu.VMEM((m_total, n), jnp.bfloat16),
            pltpu.SemaphoreType.DMA((3,)),
            pltpu.SemaphoreType.DMA((3,)),
        ],
        compiler_params=pltpu.CompilerParams(collective_id=0),
    )(x)


# device time: 51583 ns/iter; 2.1955x vs baseline; 1.0004x over previous
import jax
import jax.numpy as jnp
from jax import lax
from jax.experimental import pallas as pl
from jax.experimental.pallas import tpu as pltpu

N_DEV = 4


def _ce_stage(val, j, row, asc):
    length = val.shape[0]
    down = jnp.concatenate([val[j:], val[:j]], axis=0)
    up = jnp.concatenate([val[length - j:], val[:length - j]], axis=0)
    upper = (row & j) != 0
    pval = jnp.where(upper, up, down)
    take_min = upper != asc
    return jnp.where(take_min, jnp.minimum(val, pval), jnp.maximum(val, pval))


def _bitonic_merge(val, k, row, asc):
    j = k // 2
    while j >= 1:
        val = _ce_stage(val, j, row, asc)
        j //= 2
    return val


def kernel(x):
    m_per, n = x.shape
    m_total = N_DEV * m_per

    def body(x_ref, out_ref, gath_ref, send_sems, recv_sems):
        my = lax.axis_index("i")
        left = (my - 1) % N_DEV
        right = (my + 1) % N_DEV

        val = x_ref[...].astype(jnp.bfloat16)
        lrow = lax.broadcasted_iota(jnp.int32, (m_per, 1), 0) + my * m_per
        k = 2
        while k <= m_per:
            val = _bitonic_merge(val, k, lrow, (lrow & k) == 0)
            k *= 2
        gath_ref[pl.ds(my * m_per, m_per), :] = val

        barrier_sem = pltpu.get_barrier_semaphore()
        for nbr in [left, right]:
            pl.semaphore_signal(
                barrier_sem, inc=1,
                device_id=(nbr,), device_id_type=pl.DeviceIdType.MESH,
            )
        pl.semaphore_wait(barrier_sem, 2)

        def copy(src_slot, dst_slot, sem_idx, target):
            return pltpu.make_async_remote_copy(
                src_ref=gath_ref.at[pl.ds(src_slot * m_per, m_per)],
                dst_ref=gath_ref.at[pl.ds(dst_slot * m_per, m_per)],
                send_sem=send_sems.at[sem_idx],
                recv_sem=recv_sems.at[sem_idx],
                device_id=(target,),
                device_id_type=pl.DeviceIdType.MESH,
            )

        send_r = copy(my, my, 0, right)
        send_r.start()
        send_l = copy(my, my, 1, left)
        send_l.start()

        recv_l = copy(left, left, 0, left)
        recv_l.wait_recv()
        fwd = copy(left, left, 2, right)
        fwd.start()
        recv_r = copy(right, right, 1, right)
        recv_r.wait_recv()

        p0 = my - (my % 2)
        q0 = (p0 + 2) % N_DEV
        half = 2 * m_per
        hrow = lax.broadcasted_iota(jnp.int32, (half, 1), 0)
        own_asc = p0 < 2
        own = gath_ref[pl.ds(p0 * m_per, half), :]
        own = _bitonic_merge(own, half, hrow, own_asc)

        recv_d = copy((my - 2) % N_DEV, (my - 2) % N_DEV, 2, left)
        recv_d.wait_recv()
        other = gath_ref[pl.ds(q0 * m_per, half), :]
        other = _bitonic_merge(other, half, hrow, jnp.logical_not(own_asc))

        block_a = jnp.where(own_asc, own, other)
        block_b = jnp.where(own_asc, other, own)
        val = jnp.concatenate([block_a, block_b], axis=0)
        frow = lax.broadcasted_iota(jnp.int32, (m_total, 1), 0)
        val = _bitonic_merge(val, m_total, frow, True)

        gath_ref[...] = val
        out_ref[...] = gath_ref[pl.ds(my * m_per, m_per), :].astype(jnp.float32)

        send_r.wait_send()
        send_l.wait_send()
        fwd.wait_send()

    return pl.pallas_call(
        body,
        out_shape=jax.ShapeDtypeStruct((m_per, n), jnp.float32),
        in_specs=[pl.BlockSpec(memory_space=pltpu.VMEM)],
        out_specs=pl.BlockSpec(memory_space=pltpu.VMEM),
        scratch_shapes=[
            pltpu.VMEM((m_total, n), jnp.bfloat16),
            pltpu.SemaphoreType.DMA((3,)),
            pltpu.SemaphoreType.DMA((3,)),
        ],
        compiler_params=pltpu.CompilerParams(collective_id=0),
    )(x)


# device time: 41237 ns/iter; 2.7463x vs baseline; 1.2509x over previous
import numpy as np

import jax
import jax.numpy as jnp
from jax import lax
from jax.experimental import pallas as pl
from jax.experimental.pallas import tpu as pltpu

N_DEV = 4
_BLOCK_MIN_J = 16


def _bitonic_merge(val, k, base):
    length, n = val.shape
    j = k // 2
    while j >= _BLOCK_MIN_J:
        nb = length // (2 * j)
        v = val.reshape(nb, 2, j, n)
        a, b = v[:, 0], v[:, 1]
        lo, hi = jnp.minimum(a, b), jnp.maximum(a, b)
        static_dirs = (
            ((np.arange(nb) * 2 * j + base) & k) == 0
            if isinstance(base, int)
            else None
        )
        if static_dirs is not None and static_dirs.all():
            first, second = lo, hi
        elif static_dirs is not None and not static_dirs.any():
            first, second = hi, lo
        else:
            bid = lax.broadcasted_iota(jnp.int32, (nb, 1, 1), 0)
            ab = ((bid * (2 * j) + base) & k) == 0
            first = jnp.where(ab, lo, hi)
            second = jnp.where(ab, hi, lo)
        val = jnp.stack([first, second], axis=1).reshape(length, n)
        j //= 2
    if j >= 1:
        row1 = lax.broadcasted_iota(jnp.int32, (length, 1), 0)
        asc = ((row1 + base) & k) == 0
        while j >= 1:
            down = jnp.concatenate([val[j:], val[:j]], axis=0)
            up = jnp.concatenate([val[length - j:], val[:length - j]], axis=0)
            upper = (row1 & j) != 0
            pval = jnp.where(upper, up, down)
            take_min = upper != asc
            val = jnp.where(
                take_min, jnp.minimum(val, pval), jnp.maximum(val, pval)
            )
            j //= 2
    return val


def kernel(x):
    m_per, n = x.shape
    m_total = N_DEV * m_per

    def body(x_ref, out_ref, gath_ref, send_sems, recv_sems):
        my = lax.axis_index("i")
        left = (my - 1) % N_DEV
        right = (my + 1) % N_DEV

        val = x_ref[...].astype(jnp.bfloat16)
        k = 2
        while k <= m_per:
            base = 0 if k < m_per else my * m_per
            val = _bitonic_merge(val, k, base)
            k *= 2
        gath_ref[pl.ds(my * m_per, m_per), :] = val

        barrier_sem = pltpu.get_barrier_semaphore()
        for nbr in [left, right]:
            pl.semaphore_signal(
                barrier_sem, inc=1,
                device_id=(nbr,), device_id_type=pl.DeviceIdType.MESH,
            )
        pl.semaphore_wait(barrier_sem, 2)

        def copy(slot, sem_idx, target):
            return pltpu.make_async_remote_copy(
                src_ref=gath_ref.at[pl.ds(slot * m_per, m_per)],
                dst_ref=gath_ref.at[pl.ds(slot * m_per, m_per)],
                send_sem=send_sems.at[sem_idx],
                recv_sem=recv_sems.at[sem_idx],
                device_id=(target,),
                device_id_type=pl.DeviceIdType.MESH,
            )

        send_r = copy(my, 0, right)
        send_r.start()
        send_l = copy(my, 1, left)
        send_l.start()

        recv_l = copy(left, 0, left)
        recv_l.wait_recv()
        fwd = copy(left, 2, right)
        fwd.start()
        recv_r = copy(right, 1, right)
        recv_r.wait_recv()

        p0 = my - (my % 2)
        q0 = (p0 + 2) % N_DEV
        half = 2 * m_per
        own = gath_ref[pl.ds(p0 * m_per, half), :]
        own = _bitonic_merge(own, half, p0 * m_per)

        recv_d = copy((my - 2) % N_DEV, 2, left)
        recv_d.wait_recv()
        other = gath_ref[pl.ds(q0 * m_per, half), :]
        other = _bitonic_merge(other, half, q0 * m_per)

        own_asc = p0 < 2
        block_a = jnp.where(own_asc, own, other)
        block_b = jnp.where(own_asc, other, own)
        val = jnp.concatenate([block_a, block_b], axis=0)
        val = _bitonic_merge(val, m_total, 0)

        gath_ref[...] = val
        out_ref[...] = gath_ref[pl.ds(my * m_per, m_per), :].astype(jnp.float32)

        send_r.wait_send()
        send_l.wait_send()
        fwd.wait_send()

    return pl.pallas_call(
        body,
        out_shape=jax.ShapeDtypeStruct((m_per, n), jnp.float32),
        in_specs=[pl.BlockSpec(memory_space=pltpu.VMEM)],
        out_specs=pl.BlockSpec(memory_space=pltpu.VMEM),
        scratch_shapes=[
            pltpu.VMEM((m_total, n), jnp.bfloat16),
            pltpu.SemaphoreType.DMA((3,)),
            pltpu.SemaphoreType.DMA((3,)),
        ],
        compiler_params=pltpu.CompilerParams(collective_id=0),
    )(x)
